# baseline (speedup 1.0000x reference)
import functools

import jax
import jax.numpy as jnp
from jax import lax
from jax.experimental import pallas as pl
from jax.experimental.pallas import tpu as pltpu
from jax.experimental.pallas import tpu_sc as plsc

C_IN = 64
N_Y = 496
N_X = 432
N_BATCH = 4
N_VOX = 100000
SP = N_Y * N_X
N_SPATIAL = N_BATCH * SP
N_WORKERS = 32
SLOTS = N_SPATIAL // N_WORKERS
RANGES_PER_IMG = SP // SLOTS
N_OUT = C_IN * N_SPATIAL
N_XP = 512
TROWS = N_Y // 8
PSP = TROWS * 8 * N_XP
N_PHYS = N_BATCH * C_IN * PSP
ZCHUNK = 31744
ZPER = N_PHYS // N_WORKERS // ZCHUNK
CHUNK = 2000
N_CHUNKS = N_VOX // CHUNK
VECS_PER_CHUNK = CHUNK // 16
SLOT_VECS = SLOTS // 16
GCHUNK = 128
KDEPTH = 8

_mesh = plsc.VectorSubcoreMesh(core_axis_name="c", subcore_axis_name="s")


@functools.partial(
    pl.kernel,
    mesh=_mesh,
    compiler_params=pltpu.CompilerParams(needs_layout_passes=False,
                                         use_tc_tiling_on_sc=True),
    out_type=jax.ShapeDtypeStruct((N_PHYS,), jnp.float32),
    scratch_types=[
        pltpu.VMEM((SLOTS + 128,), jnp.int32),
        pltpu.VMEM((SLOTS + 128,), jnp.int32),
        pltpu.VMEM((ZCHUNK,), jnp.float32),
        pltpu.VMEM((2 * CHUNK,), jnp.int32),
        pltpu.VMEM((2 * CHUNK,), jnp.int32),
        pltpu.VMEM((2 * CHUNK,), jnp.int32),
        pltpu.VMEM((KDEPTH, GCHUNK), jnp.int32),
        pltpu.VMEM((KDEPTH, GCHUNK), jnp.int32),
        pltpu.VMEM((KDEPTH, GCHUNK), jnp.float32),
        pltpu.VMEM((16,), jnp.int32),
        pltpu.SemaphoreType.DMA,
        pltpu.SemaphoreType.DMA,
        pltpu.SemaphoreType.DMA,
        pltpu.SemaphoreType.DMA,
        pltpu.SemaphoreType.DMA,
    ],
)
def _sc_scatter(feat_hbm, c0_hbm, c2_hbm, c3_hbm, bs_hbm, zeros_hbm, out_hbm,
                winner_v, lslot_v, zero_v, c0r, c2r, c3r,
                gidx_v, sidx_v, gval_v, bs_v,
                sem_z, sem_c0, sem_c1, sem_g, sem_s):
    wid = lax.axis_index("c") * 16 + lax.axis_index("s")
    bnum = wid // RANGES_PER_IMG
    sp_base = (wid % RANGES_PER_IMG) * SLOTS
    lo = wid * SLOTS
    iota = lax.iota(jnp.int32, 16)

    pltpu.sync_copy(bs_hbm, bs_v)
    bs_vec = bs_v[...]

    import contextlib
    with jax.named_scope("ph_init"):
        pltpu.sync_copy(zeros_hbm, zero_v)

    def _fill(i, _):
        winner_v[pl.ds(i * 16, 16)] = jnp.full((16,), -1, jnp.int32)
        return 0
    with jax.named_scope("ph_fill"):
        lax.fori_loop(0, SLOT_VECS, _fill, 0)

    z0 = wid * (N_PHYS // N_WORKERS)
    def _fire_zero(zi, _):
        pltpu.async_copy(zero_v, out_hbm.at[pl.ds(z0 + zi * ZCHUNK, ZCHUNK)], sem_z)
        return 0
    with jax.named_scope("ph_fire_zero"):
        lax.fori_loop(0, ZPER, _fire_zero, 0)

    def _start_chunk(j, slot):
        dst = slot * CHUNK
        sem = sem_c0 if slot == 0 else sem_c1
        pltpu.async_copy(c0_hbm.at[pl.ds(j * CHUNK, CHUNK)], c0r.at[pl.ds(dst, CHUNK)], sem)
        pltpu.async_copy(c2_hbm.at[pl.ds(j * CHUNK, CHUNK)], c2r.at[pl.ds(dst, CHUNK)], sem)
        pltpu.async_copy(c3_hbm.at[pl.ds(j * CHUNK, CHUNK)], c3r.at[pl.ds(dst, CHUNK)], sem)

    def _wait_chunk(slot):
        dst = slot * CHUNK
        sem = sem_c0 if slot == 0 else sem_c1
        pltpu.make_async_copy(c0_hbm.at[pl.ds(0, CHUNK)], c0r.at[pl.ds(dst, CHUNK)], sem).wait()
        pltpu.make_async_copy(c2_hbm.at[pl.ds(0, CHUNK)], c2r.at[pl.ds(dst, CHUNK)], sem).wait()
        pltpu.make_async_copy(c3_hbm.at[pl.ds(0, CHUNK)], c3r.at[pl.ds(dst, CHUNK)], sem).wait()

    _start_chunk(0, 0)

    def _scan_pair(k, _):
        for s in (0, 1):
            j = k * 2 + s
            _wait_chunk(s)

            @pl.when(j + 1 < N_CHUNKS)
            def _():
                _start_chunk(j + 1, 1 - s)

            base = s * CHUNK
            def _scan_vec(i, _, j=j, base=base):
                for u in range(5):
                    off = base + i * 80 + u * 16
                    c0 = c0r[pl.ds(off, 16)]
                    c2 = c2r[pl.ds(off, 16)]
                    c3 = c3r[pl.ds(off, 16)]
                    loc = c0 * SP + c2 * N_X + c3 - lo
                    valid = (c0 < bs_vec) & (loc >= 0) & (loc < SLOTS)
                    locc = jnp.where(valid, loc, 0)
                    ords = (j * CHUNK + i * 80 + u * 16) + iota
                    plsc.store_scatter(winner_v, [locc], ords, mask=valid)
                return 0
            lax.fori_loop(0, VECS_PER_CHUNK // 5, _scan_vec, 0)
        return 0
    with jax.named_scope("ph_scan"):
        lax.fori_loop(0, N_CHUNKS // 2, _scan_pair, 0)

    def _compact(jv, cnt):
        w = winner_v[pl.ds(jv * 16, 16)]
        m = w >= 0
        plsc.store_compressed(winner_v.at[pl.ds(cnt, 16)], w, mask=m)
        plsc.store_compressed(lslot_v.at[pl.ds(cnt, 16)], jv * 16 + iota, mask=m)
        return cnt + jnp.max(plsc.all_reduce_population_count(m))
    with jax.named_scope("ph_compact"):
        cnt = lax.fori_loop(0, SLOT_VECS, _compact, jnp.int32(0))

    def _drain_zero(zi, _):
        pltpu.make_async_copy(out_hbm.at[pl.ds(0, ZCHUNK)], zero_v, sem_z).wait()
        return 0
    with jax.named_scope("ph_drain_zero"):
        lax.fori_loop(0, ZPER, _drain_zero, 0)
        plsc.subcore_barrier()

    with jax.named_scope("ph_scatter"):
        n_ent = cnt * C_IN
        nq = (n_ent + (GCHUNK - 1)) // GCHUNK
        n_blk = (nq + (KDEPTH - 1)) // KDEPTH
        img_base = bnum * C_IN

        def _per_blk(bq, _):
            q0 = bq * KDEPTH
            for u in range(KDEPTH):
                q = q0 + u

                @pl.when(q < nq)
                def _(q=q, u=u):
                    row_g = gidx_v.at[u]
                    row_s = sidx_v.at[u]
                    for t in range(GCHUNK // 16):
                        e = jnp.minimum(q * GCHUNK + t * 16 + iota, n_ent - 1)
                        m = lax.shift_right_logical(e, 6)
                        ch = e & (C_IN - 1)
                        v = plsc.load_gather(winner_v, [m])
                        l = plsc.load_gather(lslot_v, [m])
                        sp = sp_base + l
                        y = sp // N_X
                        x = sp - y * N_X
                        phys = ((img_base + ch) * TROWS +
                                lax.shift_right_logical(y, 3)) * 4096
                        phys = phys + lax.shift_left(
                            lax.shift_right_logical(x, 7), 10)
                        phys = phys + lax.shift_left(y & 7, 7) + (x & 127)
                        row_g[pl.ds(t * 16, 16)] = v * C_IN + ch
                        row_s[pl.ds(t * 16, 16)] = phys
                    pltpu.async_copy(feat_hbm.at[row_g], gval_v.at[u], sem_g)
            for u in range(KDEPTH):
                q = q0 + u

                @pl.when(q < nq)
                def _(q=q, u=u):
                    pltpu.make_async_copy(
                        feat_hbm.at[pl.ds(0, GCHUNK)], gval_v.at[u], sem_g).wait()
            for u in range(KDEPTH):
                q = q0 + u

                @pl.when(q < nq)
                def _(q=q, u=u):
                    pltpu.async_copy(gval_v.at[u], out_hbm.at[sidx_v.at[u]], sem_s)
            for u in range(KDEPTH):
                q = q0 + u

                @pl.when(q < nq)
                def _(q=q, u=u):
                    pltpu.make_async_copy(
                        gval_v.at[u], out_hbm.at[pl.ds(0, GCHUNK)], sem_s).wait()
            return 0
        lax.fori_loop(0, n_blk, _per_blk, 0)


def _fmt_body(in_ref, out_ref):
    x = in_ref[...].reshape(TROWS * 4 * 8, 128)
    for ty in range(TROWS):
        for tx in range(4):
            w = 128 if tx < 3 else N_X - 3 * 128
            r0 = (ty * 4 + tx) * 8
            out_ref[0, ty * 8:(ty + 1) * 8, tx * 128:tx * 128 + w] = (
                x[r0:r0 + 8, :w])


_fmt = pl.pallas_call(
    _fmt_body,
    grid=(N_BATCH * C_IN,),
    in_specs=[pl.BlockSpec((PSP,), lambda i: (i,))],
    out_specs=pl.BlockSpec((1, N_Y, N_X), lambda i: (i, 0, 0)),
    out_shape=jax.ShapeDtypeStruct((N_BATCH * C_IN, N_Y, N_X), jnp.float32),
)


def kernel(voxel_features, coors, batch_size):
    c0 = coors[:, 0].astype(jnp.int32)
    c2 = coors[:, 2].astype(jnp.int32)
    c3 = coors[:, 3].astype(jnp.int32)
    bs_vec = jnp.full((16,), batch_size, dtype=jnp.int32)
    zeros_src = jnp.zeros((ZCHUNK,), dtype=jnp.float32)
    out = _sc_scatter(voxel_features.reshape(-1), c0, c2, c3, bs_vec, zeros_src)
    return _fmt(out).reshape(N_BATCH, C_IN, N_Y, N_X)


# speedup vs baseline: 1.2001x; 1.2001x over previous
import functools

import jax
import jax.numpy as jnp
from jax import lax
from jax.experimental import pallas as pl
from jax.experimental.pallas import tpu as pltpu
from jax.experimental.pallas import tpu_sc as plsc

C_IN = 64
N_Y = 496
N_X = 432
N_BATCH = 4
N_VOX = 100000
SP = N_Y * N_X
N_SPATIAL = N_BATCH * SP
N_WORKERS = 32
SLOTS = N_SPATIAL // N_WORKERS
RANGES_PER_IMG = SP // SLOTS
N_XP = 512
TROWS = N_Y // 8
PSP = TROWS * 8 * N_XP
N_PHYS = N_BATCH * C_IN * PSP
ZCHUNK = 31744
ZPER = N_PHYS // N_WORKERS // ZCHUNK
CHUNK = 2000
N_CHUNKS = N_VOX // CHUNK
VECS_PER_CHUNK = CHUNK // 16
SLOT_VECS = SLOTS // 16
GCHUNK = 128
KDEPTH = 8

_mesh = plsc.VectorSubcoreMesh(core_axis_name="c", subcore_axis_name="s")


@functools.partial(
    pl.kernel,
    mesh=_mesh,
    compiler_params=pltpu.CompilerParams(needs_layout_passes=False),
    out_type=jax.ShapeDtypeStruct((N_PHYS,), jnp.float32),
    scratch_types=[
        pltpu.VMEM((SLOTS + 128,), jnp.int32),
        pltpu.VMEM((SLOTS + 128,), jnp.int32),
        pltpu.VMEM((ZCHUNK,), jnp.float32),
        pltpu.VMEM((2 * CHUNK,), jnp.int32),
        pltpu.VMEM((2 * CHUNK,), jnp.int32),
        pltpu.VMEM((2 * CHUNK,), jnp.int32),
        pltpu.VMEM((KDEPTH, GCHUNK), jnp.int32),
        pltpu.VMEM((KDEPTH, GCHUNK), jnp.int32),
        pltpu.VMEM((KDEPTH, GCHUNK), jnp.float32),
        pltpu.VMEM((16,), jnp.int32),
        pltpu.SemaphoreType.DMA,
        pltpu.SemaphoreType.DMA,
        pltpu.SemaphoreType.DMA,
        pltpu.SemaphoreType.DMA,
        pltpu.SemaphoreType.DMA,
    ],
)
def _sc_scatter(feat_hbm, c0_hbm, c2_hbm, c3_hbm, bs_hbm, zeros_hbm, out_hbm,
                winner_v, lslot_v, zero_v, c0r, c2r, c3r,
                gidx_v, sidx_v, gval_v, bs_v,
                sem_z, sem_c0, sem_c1, sem_g, sem_s):
    wid = lax.axis_index("c") * 16 + lax.axis_index("s")
    bnum = wid // RANGES_PER_IMG
    sp_base = (wid % RANGES_PER_IMG) * SLOTS
    lo = wid * SLOTS
    iota = lax.iota(jnp.int32, 16)

    pltpu.sync_copy(bs_hbm, bs_v)
    bs_vec = bs_v[...]

    pltpu.sync_copy(zeros_hbm, zero_v)

    def _fill(i, _):
        winner_v[pl.ds(i * 16, 16)] = jnp.full((16,), -1, jnp.int32)
        return 0
    lax.fori_loop(0, SLOT_VECS, _fill, 0)

    z0 = wid * (N_PHYS // N_WORKERS)

    def _fire_zero(zi, _):
        pltpu.async_copy(zero_v, out_hbm.at[pl.ds(z0 + zi * ZCHUNK, ZCHUNK)], sem_z)
        return 0
    lax.fori_loop(0, ZPER, _fire_zero, 0)

    def _start_chunk(j, slot):
        dst = slot * CHUNK
        sem = sem_c0 if slot == 0 else sem_c1
        pltpu.async_copy(c0_hbm.at[pl.ds(j * CHUNK, CHUNK)], c0r.at[pl.ds(dst, CHUNK)], sem)
        pltpu.async_copy(c2_hbm.at[pl.ds(j * CHUNK, CHUNK)], c2r.at[pl.ds(dst, CHUNK)], sem)
        pltpu.async_copy(c3_hbm.at[pl.ds(j * CHUNK, CHUNK)], c3r.at[pl.ds(dst, CHUNK)], sem)

    def _wait_chunk(slot):
        dst = slot * CHUNK
        sem = sem_c0 if slot == 0 else sem_c1
        pltpu.make_async_copy(c0_hbm.at[pl.ds(0, CHUNK)], c0r.at[pl.ds(dst, CHUNK)], sem).wait()
        pltpu.make_async_copy(c2_hbm.at[pl.ds(0, CHUNK)], c2r.at[pl.ds(dst, CHUNK)], sem).wait()
        pltpu.make_async_copy(c3_hbm.at[pl.ds(0, CHUNK)], c3r.at[pl.ds(dst, CHUNK)], sem).wait()

    _start_chunk(0, 0)

    def _scan_pair(k, _):
        for s in (0, 1):
            j = k * 2 + s
            _wait_chunk(s)

            @pl.when(j + 1 < N_CHUNKS)
            def _():
                _start_chunk(j + 1, 1 - s)

            base = s * CHUNK

            def _scan_vec(i, _, j=j, base=base):
                for u in range(5):
                    off = base + i * 80 + u * 16
                    c0 = c0r[pl.ds(off, 16)]
                    c2 = c2r[pl.ds(off, 16)]
                    c3 = c3r[pl.ds(off, 16)]
                    loc = c0 * SP + c2 * N_X + c3 - lo
                    valid = (c0 < bs_vec) & (loc >= 0) & (loc < SLOTS)
                    locc = jnp.where(valid, loc, 0)
                    ords = (j * CHUNK + i * 80 + u * 16) + iota
                    plsc.store_scatter(winner_v, [locc], ords, mask=valid)
                return 0
            lax.fori_loop(0, VECS_PER_CHUNK // 5, _scan_vec, 0)
        return 0
    lax.fori_loop(0, N_CHUNKS // 2, _scan_pair, 0)

    def _compact(jv, cnt):
        w = winner_v[pl.ds(jv * 16, 16)]
        m = w >= 0
        plsc.store_compressed(winner_v.at[pl.ds(cnt, 16)], w, mask=m)
        plsc.store_compressed(lslot_v.at[pl.ds(cnt, 16)], jv * 16 + iota, mask=m)
        return cnt + jnp.max(plsc.all_reduce_population_count(m))
    cnt = lax.fori_loop(0, SLOT_VECS, _compact, jnp.int32(0))

    def _drain_zero(zi, _):
        pltpu.make_async_copy(out_hbm.at[pl.ds(0, ZCHUNK)], zero_v, sem_z).wait()
        return 0
    lax.fori_loop(0, ZPER, _drain_zero, 0)
    plsc.subcore_barrier()

    n_ent = cnt * C_IN
    nq = (n_ent + (GCHUNK - 1)) // GCHUNK
    n_blk = (nq + (KDEPTH - 1)) // KDEPTH
    img_base = bnum * C_IN

    def _per_blk(bq, _):
        q0 = bq * KDEPTH
        for u in range(KDEPTH):
            q = q0 + u

            @pl.when(q < nq)
            def _(q=q, u=u):
                row_g = gidx_v.at[u]
                row_s = sidx_v.at[u]
                for t in range(GCHUNK // 16):
                    e = jnp.minimum(q * GCHUNK + t * 16 + iota, n_ent - 1)
                    m = lax.shift_right_logical(e, 6)
                    ch = e & (C_IN - 1)
                    v = plsc.load_gather(winner_v, [m])
                    l = plsc.load_gather(lslot_v, [m])
                    sp = sp_base + l
                    y = sp // N_X
                    x = sp - y * N_X
                    phys = ((img_base + ch) * TROWS +
                            lax.shift_right_logical(y, 3)) * 4096
                    phys = phys + lax.shift_left(
                        lax.shift_right_logical(x, 7), 10)
                    phys = phys + lax.shift_left(y & 7, 7) + (x & 127)
                    row_g[pl.ds(t * 16, 16)] = v * C_IN + ch
                    row_s[pl.ds(t * 16, 16)] = phys
                pltpu.async_copy(feat_hbm.at[row_g], gval_v.at[u], sem_g)
        for u in range(KDEPTH):
            q = q0 + u

            @pl.when(q < nq)
            def _(q=q, u=u):
                pltpu.make_async_copy(
                    feat_hbm.at[pl.ds(0, GCHUNK)], gval_v.at[u], sem_g).wait()
        for u in range(KDEPTH):
            q = q0 + u

            @pl.when(q < nq)
            def _(q=q, u=u):
                pltpu.async_copy(gval_v.at[u], out_hbm.at[sidx_v.at[u]], sem_s)
        for u in range(KDEPTH):
            q = q0 + u

            @pl.when(q < nq)
            def _(q=q, u=u):
                pltpu.make_async_copy(
                    gval_v.at[u], out_hbm.at[pl.ds(0, GCHUNK)], sem_s).wait()
        return 0
    lax.fori_loop(0, n_blk, _per_blk, 0)


def kernel(voxel_features, coors, batch_size):
    c0 = coors[:, 0].astype(jnp.int32)
    c2 = coors[:, 2].astype(jnp.int32)
    c3 = coors[:, 3].astype(jnp.int32)
    bs_vec = jnp.full((16,), batch_size, dtype=jnp.int32)
    zeros_src = jnp.zeros((ZCHUNK,), dtype=jnp.float32)
    out = _sc_scatter(voxel_features.reshape(-1), c0, c2, c3, bs_vec, zeros_src)
    out6 = out.reshape(N_BATCH, C_IN, TROWS, N_XP // 128, 8, 128)
    out4 = out6.transpose(0, 1, 2, 4, 3, 5).reshape(N_BATCH, C_IN, N_Y, N_XP)
    return out4[..., :N_X]
